# baseline (device time: 39047 ns/iter reference)
import jax
import jax.numpy as jnp
from jax import lax
from jax.experimental import pallas as pl
from jax.experimental.pallas import tpu as pltpu

N_DEV = 4
K_TAPS = 4
HALO = K_TAPS - 1
T_CHUNK = 512


def kernel(x, k):
    B, S, C = x.shape

    def body(x_hbm_ref, x_ref, k_ref, out_ref, halo_buf, send_sem, recv_sem):
        b = pl.program_id(0)
        my_pos = lax.axis_index("i")
        left = lax.rem(my_pos + N_DEV - 1, N_DEV)
        right = lax.rem(my_pos + 1, N_DEV)

        def make_rdma():
            return pltpu.make_async_remote_copy(
                src_ref=x_hbm_ref.at[:, S - HALO :, :],
                dst_ref=halo_buf,
                send_sem=send_sem,
                recv_sem=recv_sem,
                device_id=(right,),
                device_id_type=pl.DeviceIdType.MESH,
            )

        @pl.when(b == 0)
        def _():
            barrier = pltpu.get_barrier_semaphore()
            for nbr in (left, right):
                pl.semaphore_signal(
                    barrier,
                    inc=1,
                    device_id=(nbr,),
                    device_id_type=pl.DeviceIdType.MESH,
                )
            pl.semaphore_wait(barrier, 2)
            make_rdma().start()

        kv = k_ref[...].astype(jnp.bfloat16)

        def compute_chunk(c, tail):
            lo = c * T_CHUNK
            chunk = x_ref[0, lo : lo + T_CHUNK, :].astype(jnp.bfloat16)
            pad = jnp.concatenate([tail, chunk], axis=0)
            acc = chunk * kv[K_TAPS - 1][None, :]
            for t in range(K_TAPS - 1):
                acc += pad[t : t + T_CHUNK, :] * kv[t][None, :]
            out_ref[0, lo : lo + T_CHUNK, :] = acc * jax.nn.sigmoid(acc)

        for c in range(1, S // T_CHUNK):
            lo = c * T_CHUNK
            compute_chunk(c, x_ref[0, lo - HALO : lo, :].astype(jnp.bfloat16))

        @pl.when(b == 0)
        def _():
            make_rdma().wait()

            @pl.when(my_pos == 0)
            def _():
                halo_buf[...] = jnp.zeros_like(halo_buf)

        compute_chunk(0, halo_buf[b].astype(jnp.bfloat16))

    return pl.pallas_call(
        body,
        grid=(B,),
        out_shape=jax.ShapeDtypeStruct((B, S, C), jnp.bfloat16),
        in_specs=[
            pl.BlockSpec(memory_space=pl.ANY),
            pl.BlockSpec((1, S, C), lambda b: (b, 0, 0)),
            pl.BlockSpec((K_TAPS, C), lambda b: (0, 0)),
        ],
        out_specs=pl.BlockSpec((1, S, C), lambda b: (b, 0, 0)),
        scratch_shapes=[
            pltpu.VMEM((B, HALO, C), x.dtype),
            pltpu.SemaphoreType.DMA,
            pltpu.SemaphoreType.DMA,
        ],
        compiler_params=pltpu.CompilerParams(
            has_side_effects=True,
            collective_id=0,
            vmem_limit_bytes=56 * 1024 * 1024,
        ),
    )(x, x, k)


# device time: 29101 ns/iter; 1.3418x vs baseline; 1.3418x over previous
import jax
import jax.numpy as jnp
from jax import lax
from jax.experimental import pallas as pl
from jax.experimental.pallas import tpu as pltpu

N_DEV = 4
K_TAPS = 4
HALO = K_TAPS - 1
T_CHUNK = 512


def _make_halo_body(S):
    def _halo_body(x_hbm_ref, halo_ref, send_sem, recv_sem):
        my_pos = lax.axis_index("i")
        left = lax.rem(my_pos + N_DEV - 1, N_DEV)
        right = lax.rem(my_pos + 1, N_DEV)

        barrier = pltpu.get_barrier_semaphore()
        for nbr in (left, right):
            pl.semaphore_signal(
                barrier,
                inc=1,
                device_id=(nbr,),
                device_id_type=pl.DeviceIdType.MESH,
            )
        pl.semaphore_wait(barrier, 2)

        rdma = pltpu.make_async_remote_copy(
            src_ref=x_hbm_ref.at[:, S - HALO :, :],
            dst_ref=halo_ref,
            send_sem=send_sem,
            recv_sem=recv_sem,
            device_id=(right,),
            device_id_type=pl.DeviceIdType.MESH,
        )
        rdma.start()
        rdma.wait()

        @pl.when(my_pos == 0)
        def _():
            halo_ref[...] = jnp.zeros_like(halo_ref)

    return _halo_body


def _conv_body(x_ref, halo_ref, k_ref, out_ref):
    S = x_ref.shape[1]
    kv = k_ref[...].astype(jnp.bfloat16)
    for c in range(S // T_CHUNK):
        lo = c * T_CHUNK
        if c == 0:
            tail = halo_ref[0].astype(jnp.bfloat16)
        else:
            tail = x_ref[0, lo - HALO : lo, :].astype(jnp.bfloat16)
        chunk = x_ref[0, lo : lo + T_CHUNK, :].astype(jnp.bfloat16)
        pad = jnp.concatenate([tail, chunk], axis=0)
        acc = chunk * kv[K_TAPS - 1][None, :]
        for t in range(K_TAPS - 1):
            acc += pad[t : t + T_CHUNK, :] * kv[t][None, :]
        out_ref[0, lo : lo + T_CHUNK, :] = acc * jax.nn.sigmoid(acc)


def kernel(x, k):
    B, S, C = x.shape

    halo = pl.pallas_call(
        _make_halo_body(S),
        out_shape=jax.ShapeDtypeStruct((B, HALO, C), x.dtype),
        in_specs=[pl.BlockSpec(memory_space=pl.ANY)],
        out_specs=pl.BlockSpec(memory_space=pltpu.VMEM),
        scratch_shapes=[
            pltpu.SemaphoreType.DMA,
            pltpu.SemaphoreType.DMA,
        ],
        compiler_params=pltpu.CompilerParams(
            has_side_effects=True,
            collective_id=0,
        ),
    )(x)

    return pl.pallas_call(
        _conv_body,
        grid=(B,),
        out_shape=jax.ShapeDtypeStruct((B, S, C), jnp.bfloat16),
        in_specs=[
            pl.BlockSpec((1, S, C), lambda b: (b, 0, 0)),
            pl.BlockSpec((1, HALO, C), lambda b: (b, 0, 0)),
            pl.BlockSpec((K_TAPS, C), lambda b: (0, 0)),
        ],
        out_specs=pl.BlockSpec((1, S, C), lambda b: (b, 0, 0)),
        compiler_params=pltpu.CompilerParams(
            vmem_limit_bytes=56 * 1024 * 1024
        ),
    )(x, halo, k)
